# baseline (speedup 1.0000x reference)
import functools

import jax
import jax.numpy as jnp
from jax import lax
from jax.experimental import pallas as pl
from jax.experimental.pallas import tpu as pltpu
from jax.experimental.pallas import tpu_sc as plsc

N = 10000
P = 100000
L = 5
D = 128
HEADS = 4
G3 = 3 * D

NC = 2
NS = 16
NW = NC * NS

RW = 20000
K = 9
PR = 20480
NH = 2
PRH = PR // NH
PPH = L * PRH
ROWSH = K * PRH

TROWS = N
BB = 512
BN = 400

CG = 96
PER_WG = ROWSH // NW
NBG = 3
NCHG = PER_WG // CG

CS = 64
PER_T = PPH // NS
S_CH = PER_T // CS
PER_TS = PPH // 2 // NS
SS_CH = PER_TS // CS


def _gather_body(x_hbm, idx_hbm, out_hbm, idx_all, rows_r, gsems, wsems):
    c = lax.axis_index("c")
    s = lax.axis_index("s")
    wid = s * NC + c
    base = wid * PER_WG
    pltpu.sync_copy(idx_hbm.at[pl.ds(base, PER_WG)], idx_all)

    def outer(g, carry):
        m = g * NBG
        for b in range(NBG):
            @pl.when(g > 0)
            def _wb_done():
                pltpu.make_async_copy(
                    rows_r.at[b], out_hbm.at[pl.ds(base, CG)],
                    wsems.at[b]).wait()

            pltpu.async_copy(
                x_hbm.at[idx_all.at[pl.ds((m + b) * CG, CG)]],
                rows_r.at[b], gsems.at[b])
        for b in range(NBG):
            pltpu.make_async_copy(
                x_hbm.at[pl.ds(0, CG)], rows_r.at[b], gsems.at[b]).wait()
            pltpu.async_copy(
                rows_r.at[b], out_hbm.at[pl.ds(base + (m + b) * CG, CG)],
                wsems.at[b])
        return carry

    lax.fori_loop(0, NCHG // NBG, outer, 0)
    for b in range(NBG):
        pltpu.make_async_copy(
            rows_r.at[b], out_hbm.at[pl.ds(base, CG)], wsems.at[b]).wait()


_gather = pl.kernel(
    _gather_body,
    out_type=jax.ShapeDtypeStruct((ROWSH, D), jnp.float32),
    mesh=plsc.VectorSubcoreMesh(core_axis_name="c", subcore_axis_name="s"),
    scratch_types=[
        pltpu.VMEM((PER_WG,), jnp.int32),
        pltpu.VMEM((NBG, CG, D), jnp.float32),
        pltpu.SemaphoreType.DMA((NBG,)),
        pltpu.SemaphoreType.DMA((NBG,)),
    ],
)


def _gru_body(rw_lim, g_ref, wih_ref, whh_ref, bih_ref, bhh_ref, a_ref,
              emb_ref, att_ref):
    wih = wih_ref[...]
    whh = whh_ref[...]
    bih = bih_ref[...]
    bhh = bhh_ref[...]
    dn = (((1,), (1,)), ((), ()))
    xp = [lax.dot_general(g_ref[k].astype(jnp.float32), wih, dn,
                          preferred_element_type=jnp.float32) + bih
          for k in range(K)]
    h = jnp.zeros((L * BB, D), dtype=jnp.float32)
    for t in range(L):
        gi = jnp.concatenate([xp[4 - t + j] for j in range(L)], axis=0)
        gh = lax.dot_general(h, whh, dn, preferred_element_type=jnp.float32) + bhh
        r = jax.nn.sigmoid(gi[:, :D] + gh[:, :D])
        z = jax.nn.sigmoid(gi[:, D:2 * D] + gh[:, D:2 * D])
        n = jnp.tanh(gi[:, 2 * D:] + r * gh[:, 2 * D:])
        h = (1.0 - z) * n + z * h
    logit = lax.dot_general(h, a_ref[...], (((1,), (0,)), ((), ())),
                            preferred_element_type=jnp.float32)
    att = jnp.exp(jnp.where(logit >= 0, logit, 0.2 * logit))
    rowv = (lax.broadcasted_iota(jnp.int32, (BB, 1), 0)
            + pl.program_id(0) * BB)
    maskf = (rowv < rw_lim).astype(jnp.float32)
    att = att * jnp.concatenate([maskf] * L, axis=0)
    for j in range(L):
        for q in range(2):
            emb_ref[q, j] = h[j * BB:(j + 1) * BB, q * 64:(q + 1) * 64]
        for hd in range(HEADS):
            att_ref[j, :, hd * 16:(hd + 1) * 16] = jnp.broadcast_to(
                att[j * BB:(j + 1) * BB, hd:hd + 1], (BB, 16))


def _make_gru(rw_lim):
    return pl.pallas_call(
        functools.partial(_gru_body, rw_lim),
        grid=(PRH // BB,),
        in_specs=[
            pl.BlockSpec((K, BB, D), lambda i: (0, i, 0)),
            pl.BlockSpec((G3, D), lambda i: (0, 0)),
            pl.BlockSpec((G3, D), lambda i: (0, 0)),
            pl.BlockSpec((1, G3), lambda i: (0, 0)),
            pl.BlockSpec((1, G3), lambda i: (0, 0)),
            pl.BlockSpec((D, 8), lambda i: (0, 0)),
        ],
        out_specs=[
            pl.BlockSpec((2, L, BB, D // 2), lambda i: (0, 0, i, 0)),
            pl.BlockSpec((L, BB, HEADS * 16), lambda i: (0, i, 0)),
        ],
        out_shape=[
            jax.ShapeDtypeStruct((2, L, PRH, D // 2), jnp.float32),
            jax.ShapeDtypeStruct((L, PRH, HEADS * 16), jnp.float32),
        ],
    )


_gru_h = [_make_gru(RW - h * PRH) for h in range(NH)]


def _scatter_body(emb_hbm, att_hbm, dst_hbm, zeros_hbm, out_hbm,
                  table, ehalf_r, w_r, att_a, att_b, idx_r, lsems, ssems):
    c = lax.axis_index("c")
    s = lax.axis_index("s")
    lane = lax.iota(jnp.int32, 16)
    att_bufs = (att_a, att_b)
    hoff0 = c * 32
    hoff1 = c * 32 + 16

    for q in range(2):

        @pl.when(s == 0)
        def _zero():
            pltpu.sync_copy(zeros_hbm, table)

        plsc.subcore_barrier()

        def issue_loads(k, b):
            off = s * PER_T + k * CS
            pltpu.async_copy(
                emb_hbm.at[q, pl.ds(off, CS)],
                ehalf_r.at[b], lsems.at[b])
            pltpu.async_copy(att_hbm.at[pl.ds(off * 64, CS * 64)],
                             att_bufs[b], lsems.at[b])
            pltpu.async_copy(dst_hbm.at[pl.ds(off, CS)], idx_r.at[b],
                             lsems.at[b])

        def wait_loads(b):
            pltpu.make_async_copy(
                emb_hbm.at[0, pl.ds(0, CS)], ehalf_r.at[b],
                lsems.at[b]).wait()
            pltpu.make_async_copy(att_hbm.at[pl.ds(0, CS * 64)],
                                  att_bufs[b], lsems.at[b]).wait()
            pltpu.make_async_copy(dst_hbm.at[pl.ds(0, CS)], idx_r.at[b],
                                  lsems.at[b]).wait()

        def wait_scatter(b):
            pltpu.make_async_copy(w_r.at[b], table.at[idx_r.at[b]],
                                  ssems.at[b]).wait()

        issue_loads(0, 0)

        def outer(g, carry):
            for b in range(2):
                k = 2 * g + b
                wait_loads(b)
                if b == 1:
                    wait_scatter(0)

                    @pl.when(g < S_CH // 2 - 1)
                    def _next():
                        issue_loads(2 * g + 2, 0)
                else:
                    @pl.when(g > 0)
                    def _free():
                        wait_scatter(1)

                    issue_loads(k + 1, 1)

                def row(r, rc):
                    av0 = att_bufs[b][pl.ds(r * 64 + hoff0, 16)]
                    av1 = att_bufs[b][pl.ds(r * 64 + hoff1, 16)]
                    for q4 in range(4):
                        e = ehalf_r[b, r, pl.ds(q4 * 16, 16)]
                        w_r[b, r, pl.ds(q4 * 16, 16)] = e * av0
                        w_r[b, r, pl.ds(64 + q4 * 16, 16)] = e * av1
                    return rc

                lax.fori_loop(0, CS, row, 0)
                pltpu.async_copy(w_r.at[b], table.at[idx_r.at[b]],
                                 ssems.at[b], add=True)
            return carry

        lax.fori_loop(0, S_CH // 2, outer, 0)
        wait_scatter(1)
        plsc.subcore_barrier()

        @pl.when(s == 0)
        def _flush():
            pltpu.sync_copy(table, out_hbm.at[c, q])

        plsc.subcore_barrier()

    def zrow(r, carry):
        for b in range(2):
            for q in range(1, D // 16):
                w_r[b, r, pl.ds(q * 16, 16)] = jnp.zeros((16,), jnp.float32)
        return carry

    lax.fori_loop(0, CS, zrow, 0)
    masks = [jnp.maximum(1 - jnp.abs(lane - h), 0).astype(jnp.float32)
             for h in range(HEADS)]

    @pl.when(s == 0)
    def _zero_s():
        pltpu.sync_copy(zeros_hbm, table)

    plsc.subcore_barrier()
    sbase = c * (PPH // 2) + s * PER_TS
    att_bufs2 = (att_a, att_b)

    def sissue(k, b):
        pltpu.async_copy(att_hbm.at[pl.ds((sbase + k * CS) * 64, CS * 64)],
                         att_bufs2[b], lsems.at[b])
        pltpu.async_copy(dst_hbm.at[pl.ds(sbase + k * CS, CS)], idx_r.at[b],
                         lsems.at[b])

    def swait_loads(b):
        pltpu.make_async_copy(att_hbm.at[pl.ds(0, CS * 64)], att_bufs2[b],
                              lsems.at[b]).wait()
        pltpu.make_async_copy(dst_hbm.at[pl.ds(0, CS)], idx_r.at[b],
                              lsems.at[b]).wait()

    def swait_scatter(b):
        pltpu.make_async_copy(w_r.at[b], table.at[idx_r.at[b]],
                              ssems.at[b]).wait()

    def srows(b):
        def row(r, rc):
            acc = att_bufs2[b][pl.ds(r * 64, 16)] * masks[0]
            for h in range(1, HEADS):
                acc = acc + att_bufs2[b][pl.ds(r * 64 + h * 16, 16)] * masks[h]
            w_r[b, r, pl.ds(0, 16)] = acc
            return rc

        lax.fori_loop(0, CS, row, 0)

    sissue(0, 0)

    def souter(g, carry):
        for b in range(2):
            k = 2 * g + b
            swait_loads(b)
            if b == 1:
                swait_scatter(0)
                sissue(2 * g + 2, 0)
            else:
                @pl.when(g > 0)
                def _sfree():
                    swait_scatter(1)

                sissue(k + 1, 1)
            srows(b)
            pltpu.async_copy(w_r.at[b], table.at[idx_r.at[b]],
                             ssems.at[b], add=True)
        return carry

    lax.fori_loop(0, SS_CH // 2, souter, 0)
    swait_loads(0)
    swait_scatter(1)
    srows(0)
    pltpu.async_copy(w_r.at[0], table.at[idx_r.at[0]], ssems.at[0],
                     add=True)
    swait_scatter(0)
    plsc.subcore_barrier()

    @pl.when(s == 0)
    def _flush_s():
        pltpu.sync_copy(table, out_hbm.at[c, 2])


_scatter = pl.kernel(
    _scatter_body,
    out_type=jax.ShapeDtypeStruct((NC, 3, TROWS, D), jnp.float32),
    mesh=plsc.VectorSubcoreMesh(core_axis_name="c", subcore_axis_name="s"),
    scratch_types=[
        pltpu.VMEM_SHARED((TROWS, D), jnp.float32),
        pltpu.VMEM((2, CS, D // 2), jnp.float32),
        pltpu.VMEM((2, CS, D), jnp.float32),
        pltpu.VMEM((CS * 64,), jnp.float32),
        pltpu.VMEM((CS * 64,), jnp.float32),
        pltpu.VMEM((2, CS), jnp.int32),
        pltpu.SemaphoreType.DMA((2,)),
        pltpu.SemaphoreType.DMA((2,)),
    ],
)


def _norm_body(t0_ref, t1_ref, out_ref):
    ssum = (t0_ref[0, 2] + t0_ref[1, 2]
            + t1_ref[0, 2] + t1_ref[1, 2])
    for c in range(NC):
        for p in range(2):
            h = 2 * c + p
            for q in range(2):
                u = (t0_ref[c, q, :, p * 64:(p + 1) * 64]
                     + t1_ref[c, q, :, p * 64:(p + 1) * 64])
                lo = h * D + q * 64
                out_ref[:, lo:lo + 64] = u / ssum[:, h:h + 1]


_norm = pl.pallas_call(
    _norm_body,
    grid=(N // BN,),
    in_specs=[
        pl.BlockSpec((NC, 3, BN, D), lambda i: (0, 0, i, 0)),
        pl.BlockSpec((NC, 3, BN, D), lambda i: (0, 0, i, 0)),
    ],
    out_specs=pl.BlockSpec((BN, HEADS * D), lambda i: (i, 0)),
    out_shape=jax.ShapeDtypeStruct((N, HEADS * D), jnp.float32),
)


@jax.jit
def kernel(x, path_list, W_ih, W_hh, b_ih, b_hh, a):
    pl32 = path_list.astype(jnp.int32)
    idx2 = jnp.zeros((K, PR), jnp.int32)
    idx2 = idx2.at[:L, :RW].set(pl32[:RW, ::-1].T)
    idx2 = idx2.at[L:, :RW].set(pl32[RW:, 0].reshape(L - 1, RW))
    dstp = jnp.zeros((L, PR), jnp.int32)
    dstp = dstp.at[:, :RW].set(pl32[:, L - 1].reshape(L, RW))
    bih = b_ih.reshape(1, G3)
    bhh = b_hh.reshape(1, G3)
    apad = jnp.pad(a, ((0, 0), (0, 8 - HEADS)))
    zeros = jnp.zeros((TROWS, D), jnp.float32)

    tables = []
    for h in range(NH):
        idxh = idx2[:, h * PRH:(h + 1) * PRH].reshape(ROWSH)
        g = _gather(x, idxh)
        emb, att = _gru_h[h](g.reshape(K, PRH, D), W_ih, W_hh, bih, bhh, apad)
        dsth = dstp[:, h * PRH:(h + 1) * PRH].reshape(PPH)
        tables.append(_scatter(emb.reshape(2, PPH, D // 2),
                               att.reshape(PPH * HEADS * 16), dsth, zeros))
    return _norm(tables[0], tables[1])


# speedup vs baseline: 1.0204x; 1.0204x over previous
import functools

import jax
import jax.numpy as jnp
from jax import lax
from jax.experimental import pallas as pl
from jax.experimental.pallas import tpu as pltpu
from jax.experimental.pallas import tpu_sc as plsc

N = 10000
P = 100000
L = 5
D = 128
HEADS = 4
G3 = 3 * D

NC = 2
NS = 16
NW = NC * NS

RW = 20000
K = 9
PR = 20480
NH = 2
PRH = PR // NH
PPH = L * PRH
ROWSH = K * PRH

TROWS = N
BB = 512
BN = 400

CG = 96
PER_WG = ROWSH // NW
NBG = 3
NCHG = PER_WG // CG

CS = 64
PER_T = PPH // NS
S_CH = PER_T // CS
PER_TS = PPH // 2 // NS
SS_CH = PER_TS // CS


def _gather_body(x_hbm, idx_hbm, out_hbm, idx_all, rows_r, gsems, wsems):
    c = lax.axis_index("c")
    s = lax.axis_index("s")
    wid = s * NC + c
    base = wid * PER_WG
    pltpu.sync_copy(idx_hbm.at[pl.ds(base, PER_WG)], idx_all)

    def outer(g, carry):
        m = g * NBG
        for b in range(NBG):
            @pl.when(g > 0)
            def _wb_done():
                pltpu.make_async_copy(
                    rows_r.at[b], out_hbm.at[pl.ds(base, CG)],
                    wsems.at[b]).wait()

            pltpu.async_copy(
                x_hbm.at[idx_all.at[pl.ds((m + b) * CG, CG)]],
                rows_r.at[b], gsems.at[b])
        for b in range(NBG):
            pltpu.make_async_copy(
                x_hbm.at[pl.ds(0, CG)], rows_r.at[b], gsems.at[b]).wait()
            pltpu.async_copy(
                rows_r.at[b], out_hbm.at[pl.ds(base + (m + b) * CG, CG)],
                wsems.at[b])
        return carry

    lax.fori_loop(0, NCHG // NBG, outer, 0)
    for b in range(NBG):
        pltpu.make_async_copy(
            rows_r.at[b], out_hbm.at[pl.ds(base, CG)], wsems.at[b]).wait()


_gather = pl.kernel(
    _gather_body,
    out_type=jax.ShapeDtypeStruct((ROWSH, D), jnp.float32),
    mesh=plsc.VectorSubcoreMesh(core_axis_name="c", subcore_axis_name="s"),
    scratch_types=[
        pltpu.VMEM((PER_WG,), jnp.int32),
        pltpu.VMEM((NBG, CG, D), jnp.float32),
        pltpu.SemaphoreType.DMA((NBG,)),
        pltpu.SemaphoreType.DMA((NBG,)),
    ],
)


def _gru_body(rw_lim, g_ref, wih_ref, whh_ref, bih_ref, bhh_ref, a_ref,
              emb_ref, att_ref):
    wih = wih_ref[...]
    whh = whh_ref[...]
    bih = bih_ref[...]
    bhh = bhh_ref[...]
    dn = (((1,), (1,)), ((), ()))
    xp = [lax.dot_general(g_ref[k].astype(jnp.float32), wih, dn,
                          preferred_element_type=jnp.float32) + bih
          for k in range(K)]
    h = jnp.zeros((L * BB, D), dtype=jnp.float32)
    for t in range(L):
        gi = jnp.concatenate([xp[4 - t + j] for j in range(L)], axis=0)
        gh = lax.dot_general(h, whh, dn, preferred_element_type=jnp.float32) + bhh
        r = jax.nn.sigmoid(gi[:, :D] + gh[:, :D])
        z = jax.nn.sigmoid(gi[:, D:2 * D] + gh[:, D:2 * D])
        n = jnp.tanh(gi[:, 2 * D:] + r * gh[:, 2 * D:])
        h = (1.0 - z) * n + z * h
    logit = lax.dot_general(h, a_ref[...], (((1,), (0,)), ((), ())),
                            preferred_element_type=jnp.float32)
    att = jnp.exp(jnp.where(logit >= 0, logit, 0.2 * logit))
    rowv = (lax.broadcasted_iota(jnp.int32, (BB, 1), 0)
            + pl.program_id(0) * BB)
    maskf = (rowv < rw_lim).astype(jnp.float32)
    att = att * jnp.concatenate([maskf] * L, axis=0)
    for j in range(L):
        emb_ref[j] = h[j * BB:(j + 1) * BB]
        for hd in range(HEADS):
            att_ref[j, :, hd * 16:(hd + 1) * 16] = jnp.broadcast_to(
                att[j * BB:(j + 1) * BB, hd:hd + 1], (BB, 16))


def _make_gru(rw_lim):
    return pl.pallas_call(
        functools.partial(_gru_body, rw_lim),
        grid=(PRH // BB,),
        in_specs=[
            pl.BlockSpec((K, BB, D), lambda i: (0, i, 0)),
            pl.BlockSpec((G3, D), lambda i: (0, 0)),
            pl.BlockSpec((G3, D), lambda i: (0, 0)),
            pl.BlockSpec((1, G3), lambda i: (0, 0)),
            pl.BlockSpec((1, G3), lambda i: (0, 0)),
            pl.BlockSpec((D, 8), lambda i: (0, 0)),
        ],
        out_specs=[
            pl.BlockSpec((L, BB, D), lambda i: (0, i, 0)),
            pl.BlockSpec((L, BB, HEADS * 16), lambda i: (0, i, 0)),
        ],
        out_shape=[
            jax.ShapeDtypeStruct((L, PRH, D), jnp.float32),
            jax.ShapeDtypeStruct((L, PRH, HEADS * 16), jnp.float32),
        ],
    )


_gru_h = [_make_gru(RW - h * PRH) for h in range(NH)]


def _scatter_body(emb_hbm, att_hbm, dst_hbm, zeros_hbm, out_hbm,
                  table, emb_r, att_a, att_b, idx_r, lsems, ssems):
    c = lax.axis_index("c")
    s = lax.axis_index("s")
    lane = lax.iota(jnp.int32, 16)
    att_bufs = (att_a, att_b)

    for p in range(2):
        hoff = (2 * c + p) * 16

        @pl.when(s == 0)
        def _zero():
            pltpu.sync_copy(zeros_hbm, table)

        plsc.subcore_barrier()

        def issue_loads(k, b):
            off = s * PER_T + k * CS
            pltpu.async_copy(emb_hbm.at[pl.ds(off, CS)], emb_r.at[b],
                             lsems.at[b])
            pltpu.async_copy(att_hbm.at[pl.ds(off * 64, CS * 64)],
                             att_bufs[b], lsems.at[b])
            pltpu.async_copy(dst_hbm.at[pl.ds(off, CS)], idx_r.at[b],
                             lsems.at[b])

        def wait_loads(b):
            pltpu.make_async_copy(emb_hbm.at[pl.ds(0, CS)], emb_r.at[b],
                                  lsems.at[b]).wait()
            pltpu.make_async_copy(att_hbm.at[pl.ds(0, CS * 64)],
                                  att_bufs[b], lsems.at[b]).wait()
            pltpu.make_async_copy(dst_hbm.at[pl.ds(0, CS)], idx_r.at[b],
                                  lsems.at[b]).wait()

        def wait_scatter(b):
            pltpu.make_async_copy(emb_r.at[b], table.at[idx_r.at[b]],
                                  ssems.at[b]).wait()

        issue_loads(0, 0)

        def outer(g, carry):
            for b in range(2):
                k = 2 * g + b
                wait_loads(b)
                if b == 1:
                    wait_scatter(0)

                    @pl.when(g < S_CH // 2 - 1)
                    def _next():
                        issue_loads(2 * g + 2, 0)
                else:
                    @pl.when(g > 0)
                    def _free():
                        wait_scatter(1)

                    issue_loads(k + 1, 1)

                def row(r, rc):
                    av = att_bufs[b][pl.ds(r * 64 + hoff, 16)]
                    for q in range(D // 16):
                        emb_r[b, r, pl.ds(q * 16, 16)] = (
                            emb_r[b, r, pl.ds(q * 16, 16)] * av)
                    return rc

                lax.fori_loop(0, CS, row, 0)
                pltpu.async_copy(emb_r.at[b], table.at[idx_r.at[b]],
                                 ssems.at[b], add=True)
            return carry

        lax.fori_loop(0, S_CH // 2, outer, 0)
        wait_scatter(1)
        plsc.subcore_barrier()

        @pl.when(s == 0)
        def _flush():
            pltpu.sync_copy(table, out_hbm.at[c, p])

        plsc.subcore_barrier()

    def zrow(r, carry):
        for b in range(2):
            for q in range(1, D // 16):
                emb_r[b, r, pl.ds(q * 16, 16)] = jnp.zeros((16,), jnp.float32)
        return carry

    lax.fori_loop(0, CS, zrow, 0)
    masks = [jnp.maximum(1 - jnp.abs(lane - h), 0).astype(jnp.float32)
             for h in range(HEADS)]

    @pl.when(s == 0)
    def _zero_s():
        pltpu.sync_copy(zeros_hbm, table)

    plsc.subcore_barrier()
    sbase = c * (PPH // 2) + s * PER_TS
    att_bufs2 = (att_a, att_b)

    def sissue(k, b):
        pltpu.async_copy(att_hbm.at[pl.ds((sbase + k * CS) * 64, CS * 64)],
                         att_bufs2[b], lsems.at[b])
        pltpu.async_copy(dst_hbm.at[pl.ds(sbase + k * CS, CS)], idx_r.at[b],
                         lsems.at[b])

    def swait_loads(b):
        pltpu.make_async_copy(att_hbm.at[pl.ds(0, CS * 64)], att_bufs2[b],
                              lsems.at[b]).wait()
        pltpu.make_async_copy(dst_hbm.at[pl.ds(0, CS)], idx_r.at[b],
                              lsems.at[b]).wait()

    def swait_scatter(b):
        pltpu.make_async_copy(emb_r.at[b], table.at[idx_r.at[b]],
                              ssems.at[b]).wait()

    def srows(b):
        def row(r, rc):
            acc = att_bufs2[b][pl.ds(r * 64, 16)] * masks[0]
            for h in range(1, HEADS):
                acc = acc + att_bufs2[b][pl.ds(r * 64 + h * 16, 16)] * masks[h]
            emb_r[b, r, pl.ds(0, 16)] = acc
            return rc

        lax.fori_loop(0, CS, row, 0)

    sissue(0, 0)

    def souter(g, carry):
        for b in range(2):
            k = 2 * g + b
            swait_loads(b)
            if b == 1:
                swait_scatter(0)
                sissue(2 * g + 2, 0)
            else:
                @pl.when(g > 0)
                def _sfree():
                    swait_scatter(1)

                sissue(k + 1, 1)
            srows(b)
            pltpu.async_copy(emb_r.at[b], table.at[idx_r.at[b]],
                             ssems.at[b], add=True)
        return carry

    lax.fori_loop(0, SS_CH // 2, souter, 0)
    swait_loads(0)
    swait_scatter(1)
    srows(0)
    pltpu.async_copy(emb_r.at[0], table.at[idx_r.at[0]], ssems.at[0],
                     add=True)
    swait_scatter(0)
    plsc.subcore_barrier()

    @pl.when(s == 0)
    def _flush_s():
        pltpu.sync_copy(table, out_hbm.at[c, 2])


_scatter = pl.kernel(
    _scatter_body,
    out_type=jax.ShapeDtypeStruct((NC, 3, TROWS, D), jnp.float32),
    mesh=plsc.VectorSubcoreMesh(core_axis_name="c", subcore_axis_name="s"),
    scratch_types=[
        pltpu.VMEM_SHARED((TROWS, D), jnp.float32),
        pltpu.VMEM((2, CS, D), jnp.float32),
        pltpu.VMEM((CS * 64,), jnp.float32),
        pltpu.VMEM((CS * 64,), jnp.float32),
        pltpu.VMEM((2, CS), jnp.int32),
        pltpu.SemaphoreType.DMA((2,)),
        pltpu.SemaphoreType.DMA((2,)),
    ],
)


def _norm_body(t0_ref, t1_ref, out_ref):
    ssum = (t0_ref[0, 2] + t0_ref[1, 2]
            + t1_ref[0, 2] + t1_ref[1, 2])
    for c in range(NC):
        for p in range(2):
            h = 2 * c + p
            u = t0_ref[c, p] + t1_ref[c, p]
            out_ref[:, h * D:(h + 1) * D] = u / ssum[:, h:h + 1]


_norm = pl.pallas_call(
    _norm_body,
    grid=(N // BN,),
    in_specs=[
        pl.BlockSpec((NC, 3, BN, D), lambda i: (0, 0, i, 0)),
        pl.BlockSpec((NC, 3, BN, D), lambda i: (0, 0, i, 0)),
    ],
    out_specs=pl.BlockSpec((BN, HEADS * D), lambda i: (i, 0)),
    out_shape=jax.ShapeDtypeStruct((N, HEADS * D), jnp.float32),
)


@jax.jit
def kernel(x, path_list, W_ih, W_hh, b_ih, b_hh, a):
    pl32 = path_list.astype(jnp.int32)
    idx2 = jnp.zeros((K, PR), jnp.int32)
    idx2 = idx2.at[:L, :RW].set(pl32[:RW, ::-1].T)
    idx2 = idx2.at[L:, :RW].set(pl32[RW:, 0].reshape(L - 1, RW))
    dstp = jnp.zeros((L, PR), jnp.int32)
    dstp = dstp.at[:, :RW].set(pl32[:, L - 1].reshape(L, RW))
    bih = b_ih.reshape(1, G3)
    bhh = b_hh.reshape(1, G3)
    apad = jnp.pad(a, ((0, 0), (0, 8 - HEADS)))
    zeros = jnp.zeros((TROWS, D), jnp.float32)

    tables = []
    for h in range(NH):
        idxh = idx2[:, h * PRH:(h + 1) * PRH].reshape(ROWSH)
        g = _gather(x, idxh)
        emb, att = _gru_h[h](g.reshape(K, PRH, D), W_ih, W_hh, bih, bhh, apad)
        dsth = dstp[:, h * PRH:(h + 1) * PRH].reshape(PPH)
        tables.append(_scatter(emb.reshape(PPH, D),
                               att.reshape(PPH * HEADS * 16), dsth, zeros))
    return _norm(tables[0], tables[1])
